# baseline (speedup 1.0000x reference)
import functools

import jax
import jax.numpy as jnp
from jax import lax
from jax.experimental import pallas as pl
from jax.experimental.pallas import tpu as pltpu
from jax.experimental.pallas import tpu_sc as plsc

B = 4096
L = 200
V = 128
D = 32
NC = 2
NS = 16
LANES = 16
NW = NC * NS
BPW = B // NW
G = BPW // LANES
HALF = BPW // 2
LP = L // 4
ROT = 3


def _sc_body(x_hbm, tab_hbm, w_hbm, b_hbm, out_hbm,
             x_vm, tab_vm, w_vm, b_vm, vrep_vm, out_vm, sem0, sem1):
    cid = lax.axis_index("c")
    sid = lax.axis_index("s")
    wid = sid * NC + cid
    base = wid * BPW

    cp0 = pltpu.async_copy(x_hbm.at[pl.ds(base, HALF)],
                           x_vm.at[pl.ds(0, HALF)], sem0)
    cp1 = pltpu.async_copy(x_hbm.at[pl.ds(base + HALF, HALF)],
                           x_vm.at[pl.ds(HALF, HALF)], sem1)
    pltpu.sync_copy(tab_hbm, tab_vm)
    pltpu.sync_copy(w_hbm, w_vm)
    pltpu.sync_copy(b_hbm, b_vm)

    lanes = lax.iota(jnp.int32, LANES)

    def vfold(c, _):
        rbase = (lanes + c * LANES) << 5
        acc = jnp.zeros((LANES,), jnp.float32)
        for d in range(D):
            dl = (lanes + d) & (D - 1)
            tval = plsc.load_gather(tab_vm, [rbase + dl])
            wval = plsc.load_gather(w_vm, [(dl << 4) + lanes])
            acc = acc + tval * wval
        rv16 = (lanes + c * LANES) << 4
        for t in range(LANES):
            plsc.store_scatter(vrep_vm, [rv16 + ((lanes + t) & (LANES - 1))],
                               acc)
        return 0

    lax.fori_loop(0, G, vfold, 0)

    rows = [lanes + g * LANES for g in range(G)]
    coloff = lanes * ROT
    zero = jnp.zeros((LANES,), jnp.float32)

    UNROLL = 2

    def make_body(glo, ghi):
        def body(jb, accs):
            j0 = jb * UNROLL
            accs = list(accs)
            for u in range(UNROLL):
                col = coloff + (j0 + u)
                col = jnp.where(col >= LP, col - LP, col)
                for i, g in enumerate(range(glo, ghi)):
                    xw = plsc.load_gather(x_vm, [rows[g], col])
                    a = accs[i]
                    a = a + plsc.load_gather(
                        vrep_vm, [((xw << 4) & 0xFF0) + lanes])
                    a = a + plsc.load_gather(
                        vrep_vm, [((xw >> 4) & 0xFF0) + lanes])
                    a = a + plsc.load_gather(
                        vrep_vm, [((xw >> 12) & 0xFF0) + lanes])
                    a = a + plsc.load_gather(
                        vrep_vm, [((xw >> 20) & 0xFF0) + lanes])
                    accs[i] = a
            return tuple(accs)
        return body

    cp0.wait()
    accs_lo = lax.fori_loop(0, L // UNROLL, make_body(0, G // 2),
                            tuple(zero for _ in range(G // 2)))
    cp1.wait()
    accs_hi = lax.fori_loop(0, L // UNROLL, make_body(G // 2, G),
                            tuple(zero for _ in range(G // 2)))

    bvec = b_vm[pl.ds(0, LANES)]
    for g, acc in enumerate(accs_lo + accs_hi):
        z = acc * (1.0 / L) + bvec
        out_vm[pl.ds(g * LANES, LANES)] = 1.0 / (1.0 + jnp.exp(-z))

    pltpu.sync_copy(out_vm, out_hbm.at[pl.ds(base, BPW)])


_tt_call = functools.partial(
    pl.kernel,
    out_type=jax.ShapeDtypeStruct((B,), jnp.float32),
    mesh=plsc.VectorSubcoreMesh(core_axis_name="c", subcore_axis_name="s"),
    compiler_params=pltpu.CompilerParams(needs_layout_passes=False,
                                         use_tc_tiling_on_sc=True),
    scratch_types=[
        pltpu.VMEM((BPW, LP), jnp.int32),
        pltpu.VMEM((V * D,), jnp.float32),
        pltpu.VMEM((D * LANES,), jnp.float32),
        pltpu.VMEM((LANES,), jnp.float32),
        pltpu.VMEM((V * LANES,), jnp.float32),
        pltpu.VMEM((BPW,), jnp.float32),
        pltpu.SemaphoreType.DMA,
        pltpu.SemaphoreType.DMA,
    ],
)(_sc_body)


def kernel(x, table, W, b):
    xp = jax.lax.bitcast_convert_type(
        x.astype(jnp.uint8).reshape(B, LP, 4), jnp.int32)
    w_rep = jnp.broadcast_to(W.reshape(D, 1), (D, LANES)).reshape(D * LANES)
    out = _tt_call(xp, table.reshape(V * D), w_rep,
                   jnp.broadcast_to(b, (LANES,)))
    return out.reshape(B, 1)


# speedup vs baseline: 1.2837x; 1.2837x over previous
import functools

import jax
import jax.numpy as jnp
from jax import lax
from jax.experimental import pallas as pl
from jax.experimental.pallas import tpu as pltpu
from jax.experimental.pallas import tpu_sc as plsc

B = 4096
L = 200
V = 128
D = 32
NC = 2
NS = 16
LANES = 16
NW = NC * NS
BPW = B // NW
G = BPW // LANES
ROT = 13


def _sc_body(x_hbm, tab_hbm, w_hbm, b_hbm, out_hbm,
             x_vm, tab_vm, w_vm, b_vm, vrep_vm, out_vm):
    cid = lax.axis_index("c")
    sid = lax.axis_index("s")
    wid = sid * NC + cid

    pltpu.sync_copy(x_hbm.at[pl.ds(wid * BPW, BPW)], x_vm)
    pltpu.sync_copy(tab_hbm, tab_vm)
    pltpu.sync_copy(w_hbm, w_vm)
    pltpu.sync_copy(b_hbm, b_vm)

    lanes = lax.iota(jnp.int32, LANES)

    w0 = w_vm[pl.ds(0, LANES)]
    w1 = w_vm[pl.ds(LANES, LANES)]
    for r in range(V):
        prod = tab_vm[pl.ds(r * D, LANES)] * w0 + \
            tab_vm[pl.ds(r * D + LANES, LANES)] * w1
        s = jnp.sum(prod, axis=0)
        vrep_vm[pl.ds(r * LANES, LANES)] = jnp.broadcast_to(s, (LANES,))

    rows = [lanes + g * LANES for g in range(G)]
    coloff = lanes * ROT
    zero16 = jnp.zeros((LANES,), jnp.int32)

    UNROLL = 8

    def body(jb, accs):
        j0 = jb * UNROLL
        accs = list(accs)
        for u in range(UNROLL):
            col = coloff + (j0 + u)
            col = jnp.where(col >= L, col - L, col)
            for g in range(G):
                xv = plsc.load_gather(x_vm, [rows[g], col])
                vidx = (xv << 4) + lanes
                accs[g] = accs[g] + plsc.load_gather(vrep_vm, [vidx])
        return tuple(accs)

    zero = jnp.zeros((LANES,), jnp.float32)
    accs = lax.fori_loop(0, L // UNROLL, body, tuple(zero for _ in range(G)))

    bvec = b_vm[...]
    for g in range(G):
        z = accs[g] * (1.0 / L) + bvec
        out_vm[pl.ds(g * LANES, LANES)] = 1.0 / (1.0 + jnp.exp(-z))

    pltpu.sync_copy(out_vm, out_hbm.at[pl.ds(wid * BPW, BPW)])


_tt_call = functools.partial(
    pl.kernel,
    out_type=jax.ShapeDtypeStruct((B,), jnp.float32),
    mesh=plsc.VectorSubcoreMesh(core_axis_name="c", subcore_axis_name="s"),
    compiler_params=pltpu.CompilerParams(needs_layout_passes=False),
    scratch_types=[
        pltpu.VMEM((BPW, L), jnp.int32),
        pltpu.VMEM((V * D,), jnp.float32),
        pltpu.VMEM((D,), jnp.float32),
        pltpu.VMEM((LANES,), jnp.float32),
        pltpu.VMEM((V * LANES,), jnp.float32),
        pltpu.VMEM((BPW,), jnp.float32),
    ],
)(_sc_body)


def kernel(x, table, W, b):
    out = _tt_call(x.astype(jnp.int32), table.reshape(V * D), W.reshape(D),
                   jnp.broadcast_to(b, (LANES,)))
    return out.reshape(B, 1)
